# baseline (speedup 1.0000x reference)
import dataclasses
import functools

import jax
import jax.numpy as jnp
from jax import lax
from jax.experimental import pallas as pl
from jax.experimental.pallas import tpu as pltpu
from jax.experimental.pallas import tpu_sc as plsc

N_NODES = 10000
DIM_IN = 128
DIM_OUT = 64
NUM_EDGES = 320000

NUM_CORES = 2
NUM_SUBCORES = 16
NW = NUM_CORES * NUM_SUBCORES

CH = 128
NCHUNK = 80
EPT = NCHUNK * CH
PAD_EDGES = NW * EPT - NUM_EDGES
NCHUNK_MP = 160
CCOL = DIM_OUT // NUM_CORES
DUMMY = N_NODES
NPAD = N_NODES + 8
HIST = N_NODES + 16
ACC_ROWS = 10240
SLAB = ACC_ROWS // NUM_SUBCORES
NPIECE = SLAB // CH

ROWS_PER_MM_BLOCK = 1000
NBUF = 8


def _vector_mesh():
    return plsc.VectorSubcoreMesh(core_axis_name="c", subcore_axis_name="s")


def _sc_compiler_params():
    cp = pltpu.CompilerParams()
    fields = pltpu.CompilerParams.__dataclass_fields__
    if "needs_layout_passes" in fields:
        cp = dataclasses.replace(cp, needs_layout_passes=False)
    if "use_tc_tiling_on_sc" in fields:
        cp = dataclasses.replace(cp, use_tc_tiling_on_sc=False)
    return cp


def _sc_degree(dst3d):
    @functools.partial(
        pl.kernel,
        out_type=jax.ShapeDtypeStruct((NW, HIST), jnp.float32),
        mesh=_vector_mesh(),
        compiler_params=_sc_compiler_params(),
        scratch_types=[
            pltpu.VMEM((NCHUNK, CH), jnp.int32),
            pltpu.VMEM((HIST,), jnp.float32),
        ],
    )
    def deg_kernel(dst_hbm, out_hbm, idx_v, hist_v):
        cid = lax.axis_index("c")
        sid = lax.axis_index("s")
        wid = cid * NUM_SUBCORES + sid

        zeros16 = jnp.zeros((16,), jnp.float32)

        @pl.loop(0, HIST, step=16)
        def _(i):
            hist_v[pl.ds(i, 16)] = zeros16

        pltpu.sync_copy(dst_hbm.at[wid], idx_v)

        ones16 = jnp.ones((16,), jnp.float32)

        @pl.loop(0, NCHUNK)
        def _(j):
            @pl.loop(0, CH, step=16)
            def _(k):
                idx = idx_v.at[j][pl.ds(k, 16)]
                plsc.addupdate_scatter(hist_v, [idx], ones16)

        pltpu.sync_copy(hist_v, out_hbm.at[wid])

    return deg_kernel(dst3d)


def _sc_scatter_half(g_half, src3d, dst3d):
    @functools.partial(
        pl.kernel,
        out_type=jax.ShapeDtypeStruct((ACC_ROWS, CCOL), jnp.float32),
        mesh=plsc.VectorSubcoreMesh(
            core_axis_name="c", subcore_axis_name="s", num_cores=1),
        compiler_params=_sc_compiler_params(),
        scratch_types=[
            pltpu.VMEM((NCHUNK_MP, CH), jnp.int32),
            pltpu.VMEM((NCHUNK_MP, CH), jnp.int32),
            pltpu.VMEM((NBUF, CH, CCOL), jnp.float32),
            pltpu.VMEM_SHARED((ACC_ROWS, CCOL), jnp.float32),
            pltpu.VMEM_SHARED((ACC_ROWS, CCOL), jnp.float32),
            pltpu.SemaphoreType.DMA((NBUF,)),
            pltpu.SemaphoreType.DMA((NBUF,)),
        ],
    )
    def mp_kernel(g_hbm, src_hbm, dst_hbm, out_hbm,
                  src_v, dst_v, rows_v, acc_sh, g_sh, semg, sems):
        sid = lax.axis_index("s")

        zeros16 = jnp.zeros((16,), jnp.float32)

        @pl.loop(0, CH)
        def _(r):
            @pl.loop(0, CCOL, step=16)
            def _(k):
                rows_v.at[0][r, pl.ds(k, 16)] = zeros16

        @pl.loop(0, NPIECE)
        def _(q):
            pltpu.sync_copy(rows_v.at[0],
                            acc_sh.at[pl.ds(sid * SLAB + q * CH, CH)])

        @pl.loop(0, NPIECE)
        def _(q):
            base = sid * SLAB + q * CH
            pltpu.sync_copy(g_hbm.at[pl.ds(base, CH)],
                            g_sh.at[pl.ds(base, CH)])

        plsc.subcore_barrier()

        pltpu.sync_copy(src_hbm.at[sid], src_v)
        pltpu.sync_copy(dst_hbm.at[sid], dst_v)

        for b in range(NBUF):
            pltpu.async_copy(g_sh.at[src_v.at[b]], rows_v.at[b], semg.at[b])

        @pl.loop(0, NCHUNK_MP, step=NBUF)
        def _(j):
            for b in range(NBUF):
                pltpu.make_async_copy(
                    g_sh.at[src_v.at[j + b]], rows_v.at[b], semg.at[b]).wait()
                pltpu.async_copy(
                    rows_v.at[b], acc_sh.at[dst_v.at[j + b]], sems.at[b],
                    add=True)
            for b in range(NBUF):
                pltpu.make_async_copy(
                    rows_v.at[b], acc_sh.at[dst_v.at[j + b]], sems.at[b]).wait()

                @pl.when(j + NBUF + b < NCHUNK_MP)
                def _(j=j, b=b):
                    pltpu.async_copy(
                        g_sh.at[src_v.at[j + NBUF + b]], rows_v.at[b],
                        semg.at[b])

        plsc.subcore_barrier()

        @pl.loop(0, NPIECE)
        def _(q):
            base = sid * SLAB + q * CH
            pltpu.sync_copy(acc_sh.at[pl.ds(base, CH)],
                            out_hbm.at[pl.ds(base, CH)])

    return mp_kernel(g_half, src3d, dst3d)


def _tc_matmul(x, W):
    def body(x_ref, w_ref, o_ref):
        o_ref[...] = jnp.dot(x_ref[...], w_ref[...],
                             preferred_element_type=jnp.float32)

    nblk = N_NODES // ROWS_PER_MM_BLOCK
    return pl.pallas_call(
        body,
        out_shape=jax.ShapeDtypeStruct((N_NODES, DIM_OUT), jnp.float32),
        grid=(nblk,),
        in_specs=[
            pl.BlockSpec((ROWS_PER_MM_BLOCK, DIM_IN), lambda i: (i, 0)),
            pl.BlockSpec((DIM_IN, DIM_OUT), lambda i: (0, 0)),
        ],
        out_specs=pl.BlockSpec((ROWS_PER_MM_BLOCK, DIM_OUT), lambda i: (i, 0)),
    )(x, W)


def _tc_degsum(partials):
    def body(p_ref, o_ref):
        o_ref[...] = jnp.sum(p_ref[...], axis=0, keepdims=True) + 2.0

    return pl.pallas_call(
        body,
        out_shape=jax.ShapeDtypeStruct((1, HIST), jnp.float32),
    )(partials)


def _tc_scale(h, deg_col):
    def body(h_ref, d_ref, o_ref):
        o_ref[...] = h_ref[...] * lax.rsqrt(d_ref[...])

    nblk = N_NODES // ROWS_PER_MM_BLOCK
    return pl.pallas_call(
        body,
        out_shape=jax.ShapeDtypeStruct((N_NODES, DIM_OUT), jnp.float32),
        grid=(nblk,),
        in_specs=[
            pl.BlockSpec((ROWS_PER_MM_BLOCK, DIM_OUT), lambda i: (i, 0)),
            pl.BlockSpec((ROWS_PER_MM_BLOCK, 1), lambda i: (i, 0)),
        ],
        out_specs=pl.BlockSpec((ROWS_PER_MM_BLOCK, DIM_OUT), lambda i: (i, 0)),
    )(h, deg_col)


def _tc_final(a0, a1, g, deg_col, b2, lw2, lb2):
    def body(a0_ref, a1_ref, g_ref, d_ref, b_ref, lw_ref, lb_ref, o_ref):
        dis = lax.rsqrt(d_ref[...])
        acc = jnp.concatenate([a0_ref[...], a1_ref[...]], axis=1)
        out = dis * (acc + 2.0 * g_ref[...]) + b_ref[...]
        out = jnp.where(out >= 0, out, 0.01 * out)
        mu = jnp.mean(out, axis=1, keepdims=True)
        cen = out - mu
        var = jnp.mean(cen * cen, axis=1, keepdims=True)
        o_ref[...] = cen * lax.rsqrt(var + 1e-5) * lw_ref[...] + lb_ref[...]

    nblk = N_NODES // ROWS_PER_MM_BLOCK
    row_spec = pl.BlockSpec((ROWS_PER_MM_BLOCK, DIM_OUT), lambda i: (i, 0))
    half_spec = pl.BlockSpec((ROWS_PER_MM_BLOCK, CCOL), lambda i: (i, 0))
    vec_spec = pl.BlockSpec((1, DIM_OUT), lambda i: (0, 0))
    return pl.pallas_call(
        body,
        out_shape=jax.ShapeDtypeStruct((N_NODES, DIM_OUT), jnp.float32),
        grid=(nblk,),
        in_specs=[half_spec, half_spec, row_spec,
                  pl.BlockSpec((ROWS_PER_MM_BLOCK, 1), lambda i: (i, 0)),
                  vec_spec, vec_spec, vec_spec],
        out_specs=row_spec,
    )(a0, a1, g, deg_col, b2, lw2, lb2)


@jax.jit
def _run(x, edge_index, W, b, ln_w, ln_b):
    src = edge_index[0]
    dst = edge_index[1]
    pad = jnp.full((PAD_EDGES,), DUMMY, jnp.int32)
    src_flat = jnp.concatenate([src, pad])
    dst_flat = jnp.concatenate([dst, pad])
    dst3d = dst_flat.reshape(NW, NCHUNK, CH)
    src3d_mp = src_flat.reshape(NUM_SUBCORES, NCHUNK_MP, CH)
    dst3d_mp = dst_flat.reshape(NUM_SUBCORES, NCHUNK_MP, CH)

    partials = _sc_degree(dst3d)
    h = _tc_matmul(x, W)
    deg_col = _tc_degsum(partials).reshape(HIST, 1)[:N_NODES]
    g = _tc_scale(h, deg_col)
    g_pad = jnp.pad(g, ((0, ACC_ROWS - N_NODES), (0, 0)))
    acc0 = _sc_scatter_half(g_pad[:, :CCOL], src3d_mp, dst3d_mp)
    acc1 = _sc_scatter_half(g_pad[:, CCOL:], src3d_mp, dst3d_mp)

    b2 = b.reshape(1, DIM_OUT)
    lw2 = ln_w.reshape(1, DIM_OUT)
    lb2 = ln_b.reshape(1, DIM_OUT)
    return _tc_final(acc0, acc1, g, deg_col, b2, lw2, lb2)


def kernel(x, edge_index, W, b, ln_w, ln_b):
    return _run(x, edge_index, W, b, ln_w, ln_b)


# speedup vs baseline: 1.3416x; 1.3416x over previous
import dataclasses
import functools

import jax
import jax.numpy as jnp
from jax import lax
from jax.experimental import pallas as pl
from jax.experimental.pallas import tpu as pltpu
from jax.experimental.pallas import tpu_sc as plsc

N_NODES = 10000
DIM_IN = 128
DIM_OUT = 64
NUM_EDGES = 320000

NUM_CORES = 2
NUM_SUBCORES = 16
NW = NUM_CORES * NUM_SUBCORES

CH = 128
NCHUNK = 80
EPT = NCHUNK * CH
PAD_EDGES = NW * EPT - NUM_EDGES
NCHUNK_MP = 160
CCOL = DIM_OUT // NUM_CORES
DUMMY = N_NODES
NPAD = N_NODES + 8
HIST = N_NODES + 16
ACC_ROWS = 10240
SLAB = ACC_ROWS // NUM_SUBCORES
NPIECE = SLAB // CH

ROWS_PER_MM_BLOCK = 1000
NBUF = 8


def _vector_mesh():
    return plsc.VectorSubcoreMesh(core_axis_name="c", subcore_axis_name="s")


def _sc_compiler_params():
    cp = pltpu.CompilerParams()
    fields = pltpu.CompilerParams.__dataclass_fields__
    if "needs_layout_passes" in fields:
        cp = dataclasses.replace(cp, needs_layout_passes=False)
    if "use_tc_tiling_on_sc" in fields:
        cp = dataclasses.replace(cp, use_tc_tiling_on_sc=False)
    return cp


def _sc_degree(dst3d):
    @functools.partial(
        pl.kernel,
        out_type=jax.ShapeDtypeStruct((NW, HIST), jnp.float32),
        mesh=_vector_mesh(),
        compiler_params=_sc_compiler_params(),
        scratch_types=[
            pltpu.VMEM((NCHUNK, CH), jnp.int32),
            pltpu.VMEM((HIST,), jnp.float32),
        ],
    )
    def deg_kernel(dst_hbm, out_hbm, idx_v, hist_v):
        cid = lax.axis_index("c")
        sid = lax.axis_index("s")
        wid = cid * NUM_SUBCORES + sid

        zeros16 = jnp.zeros((16,), jnp.float32)

        @pl.loop(0, HIST, step=16)
        def _(i):
            hist_v[pl.ds(i, 16)] = zeros16

        pltpu.sync_copy(dst_hbm.at[wid], idx_v)

        ones16 = jnp.ones((16,), jnp.float32)

        @pl.loop(0, NCHUNK)
        def _(j):
            @pl.loop(0, CH, step=16)
            def _(k):
                idx = idx_v.at[j][pl.ds(k, 16)]
                plsc.addupdate_scatter(hist_v, [idx], ones16)

        pltpu.sync_copy(hist_v, out_hbm.at[wid])

    return deg_kernel(dst3d)


def _sc_scatter(g2, src3d, dst3d):
    @functools.partial(
        pl.kernel,
        out_type=jax.ShapeDtypeStruct((NUM_CORES, ACC_ROWS, CCOL), jnp.float32),
        mesh=_vector_mesh(),
        compiler_params=_sc_compiler_params(),
        scratch_types=[
            pltpu.VMEM((NCHUNK_MP, CH), jnp.int32),
            pltpu.VMEM((NCHUNK_MP, CH), jnp.int32),
            pltpu.VMEM((NBUF, CH, CCOL), jnp.float32),
            pltpu.VMEM_SHARED((ACC_ROWS, CCOL), jnp.float32),
            pltpu.VMEM_SHARED((ACC_ROWS, CCOL), jnp.float32),
            pltpu.SemaphoreType.DMA((NBUF,)),
            pltpu.SemaphoreType.DMA((NBUF,)),
        ],
    )
    def mp_kernel(g_hbm, src_hbm, dst_hbm, out_hbm,
                  src_v, dst_v, rows_v, acc_sh, g_sh, semg, sems):
        cid = lax.axis_index("c")
        sid = lax.axis_index("s")

        zeros16 = jnp.zeros((16,), jnp.float32)

        @pl.loop(0, CH)
        def _(r):
            @pl.loop(0, CCOL, step=16)
            def _(k):
                rows_v.at[0][r, pl.ds(k, 16)] = zeros16

        @pl.loop(0, NPIECE)
        def _(q):
            pltpu.sync_copy(rows_v.at[0],
                            acc_sh.at[pl.ds(sid * SLAB + q * CH, CH)])

        @pl.loop(0, NPIECE)
        def _(q):
            base = sid * SLAB + q * CH
            pltpu.sync_copy(g_hbm.at[cid].at[pl.ds(base, CH)],
                            g_sh.at[pl.ds(base, CH)])

        plsc.subcore_barrier()

        pltpu.sync_copy(src_hbm.at[sid], src_v)
        pltpu.sync_copy(dst_hbm.at[sid], dst_v)

        for b in range(NBUF):
            pltpu.async_copy(g_sh.at[src_v.at[b]], rows_v.at[b], semg.at[b])

        @pl.loop(0, NCHUNK_MP, step=NBUF)
        def _(j):
            for b in range(NBUF):
                pltpu.make_async_copy(
                    g_sh.at[src_v.at[j + b]], rows_v.at[b], semg.at[b]).wait()
                pltpu.async_copy(
                    rows_v.at[b], acc_sh.at[dst_v.at[j + b]], sems.at[b],
                    add=True)
            for b in range(NBUF):
                pltpu.make_async_copy(
                    rows_v.at[b], acc_sh.at[dst_v.at[j + b]], sems.at[b]).wait()

                @pl.when(j + NBUF + b < NCHUNK_MP)
                def _(j=j, b=b):
                    pltpu.async_copy(
                        g_sh.at[src_v.at[j + NBUF + b]], rows_v.at[b],
                        semg.at[b])

        plsc.subcore_barrier()

        @pl.loop(0, NPIECE)
        def _(q):
            base = sid * SLAB + q * CH
            pltpu.sync_copy(acc_sh.at[pl.ds(base, CH)],
                            out_hbm.at[cid].at[pl.ds(base, CH)])

    return mp_kernel(g2, src3d, dst3d)


def _tc_matmul(x, W):
    def body(x_ref, w_ref, o_ref):
        o_ref[...] = jnp.dot(x_ref[...], w_ref[...],
                             preferred_element_type=jnp.float32)

    nblk = N_NODES // ROWS_PER_MM_BLOCK
    return pl.pallas_call(
        body,
        out_shape=jax.ShapeDtypeStruct((N_NODES, DIM_OUT), jnp.float32),
        grid=(nblk,),
        in_specs=[
            pl.BlockSpec((ROWS_PER_MM_BLOCK, DIM_IN), lambda i: (i, 0)),
            pl.BlockSpec((DIM_IN, DIM_OUT), lambda i: (0, 0)),
        ],
        out_specs=pl.BlockSpec((ROWS_PER_MM_BLOCK, DIM_OUT), lambda i: (i, 0)),
    )(x, W)


def _tc_degsum(partials):
    def body(p_ref, o_ref):
        o_ref[...] = jnp.sum(p_ref[...], axis=0, keepdims=True) + 2.0

    return pl.pallas_call(
        body,
        out_shape=jax.ShapeDtypeStruct((1, HIST), jnp.float32),
    )(partials)


def _tc_scale(h, deg_col):
    def body(h_ref, d_ref, o_ref):
        o_ref[...] = h_ref[...] * lax.rsqrt(d_ref[...])

    nblk = N_NODES // ROWS_PER_MM_BLOCK
    return pl.pallas_call(
        body,
        out_shape=jax.ShapeDtypeStruct((N_NODES, DIM_OUT), jnp.float32),
        grid=(nblk,),
        in_specs=[
            pl.BlockSpec((ROWS_PER_MM_BLOCK, DIM_OUT), lambda i: (i, 0)),
            pl.BlockSpec((ROWS_PER_MM_BLOCK, 1), lambda i: (i, 0)),
        ],
        out_specs=pl.BlockSpec((ROWS_PER_MM_BLOCK, DIM_OUT), lambda i: (i, 0)),
    )(h, deg_col)


def _tc_final(a0, a1, g, deg_col, b2, lw2, lb2):
    def body(a0_ref, a1_ref, g_ref, d_ref, b_ref, lw_ref, lb_ref, o_ref):
        dis = lax.rsqrt(d_ref[...])
        acc = jnp.concatenate([a0_ref[...], a1_ref[...]], axis=1)
        out = dis * (acc + 2.0 * g_ref[...]) + b_ref[...]
        out = jnp.where(out >= 0, out, 0.01 * out)
        mu = jnp.mean(out, axis=1, keepdims=True)
        cen = out - mu
        var = jnp.mean(cen * cen, axis=1, keepdims=True)
        o_ref[...] = cen * lax.rsqrt(var + 1e-5) * lw_ref[...] + lb_ref[...]

    nblk = N_NODES // ROWS_PER_MM_BLOCK
    row_spec = pl.BlockSpec((ROWS_PER_MM_BLOCK, DIM_OUT), lambda i: (i, 0))
    half_spec = pl.BlockSpec((ROWS_PER_MM_BLOCK, CCOL), lambda i: (i, 0))
    vec_spec = pl.BlockSpec((1, DIM_OUT), lambda i: (0, 0))
    return pl.pallas_call(
        body,
        out_shape=jax.ShapeDtypeStruct((N_NODES, DIM_OUT), jnp.float32),
        grid=(nblk,),
        in_specs=[half_spec, half_spec, row_spec,
                  pl.BlockSpec((ROWS_PER_MM_BLOCK, 1), lambda i: (i, 0)),
                  vec_spec, vec_spec, vec_spec],
        out_specs=row_spec,
    )(a0, a1, g, deg_col, b2, lw2, lb2)


@jax.jit
def _run(x, edge_index, W, b, ln_w, ln_b):
    src = edge_index[0]
    dst = edge_index[1]
    pad = jnp.full((PAD_EDGES,), DUMMY, jnp.int32)
    src_flat = jnp.concatenate([src, pad])
    dst_flat = jnp.concatenate([dst, pad])
    dst3d = dst_flat.reshape(NW, NCHUNK, CH)
    src3d_mp = src_flat.reshape(NUM_SUBCORES, NCHUNK_MP, CH)
    dst3d_mp = dst_flat.reshape(NUM_SUBCORES, NCHUNK_MP, CH)

    partials = _sc_degree(dst3d)
    h = _tc_matmul(x, W)
    deg_col = _tc_degsum(partials).reshape(HIST, 1)[:N_NODES]
    g = _tc_scale(h, deg_col)
    g_pad = jnp.pad(g, ((0, ACC_ROWS - N_NODES), (0, 0)))
    g2 = jnp.stack([g_pad[:, :CCOL], g_pad[:, CCOL:]], axis=0)
    acc = _sc_scatter(g2, src3d_mp, dst3d_mp)

    b2 = b.reshape(1, DIM_OUT)
    lw2 = ln_w.reshape(1, DIM_OUT)
    lb2 = ln_b.reshape(1, DIM_OUT)
    return _tc_final(acc[0], acc[1], g, deg_col, b2, lw2, lb2)


def kernel(x, edge_index, W, b, ln_w, ln_b):
    return _run(x, edge_index, W, b, ln_w, ln_b)


# speedup vs baseline: 1.4572x; 1.0862x over previous
import dataclasses
import functools

import jax
import jax.numpy as jnp
from jax import lax
from jax.experimental import pallas as pl
from jax.experimental.pallas import tpu as pltpu
from jax.experimental.pallas import tpu_sc as plsc

N_NODES = 10000
DIM_IN = 128
DIM_OUT = 64
NUM_EDGES = 320000

NUM_CORES = 2
NUM_SUBCORES = 16
NW = NUM_CORES * NUM_SUBCORES

CH = 128
NCHUNK = 80
EPT = NCHUNK * CH
PAD_EDGES = NW * EPT - NUM_EDGES
NCHUNK_MP = 160
CCOL = DIM_OUT // NUM_CORES
DUMMY = N_NODES
NPAD = N_NODES + 8
HIST = N_NODES + 16
ACC_ROWS = 10240
SLAB = ACC_ROWS // NUM_SUBCORES
NPIECE = SLAB // CH

ROWS_PER_MM_BLOCK = 1000
NBUF = 8
AHEAD = 4


def _vector_mesh():
    return plsc.VectorSubcoreMesh(core_axis_name="c", subcore_axis_name="s")


def _sc_compiler_params():
    cp = pltpu.CompilerParams()
    fields = pltpu.CompilerParams.__dataclass_fields__
    if "needs_layout_passes" in fields:
        cp = dataclasses.replace(cp, needs_layout_passes=False)
    if "use_tc_tiling_on_sc" in fields:
        cp = dataclasses.replace(cp, use_tc_tiling_on_sc=False)
    return cp


def _sc_degree(dst3d):
    @functools.partial(
        pl.kernel,
        out_type=jax.ShapeDtypeStruct((NW, HIST), jnp.float32),
        mesh=_vector_mesh(),
        compiler_params=_sc_compiler_params(),
        scratch_types=[
            pltpu.VMEM((NCHUNK, CH), jnp.int32),
            pltpu.VMEM((HIST,), jnp.float32),
        ],
    )
    def deg_kernel(dst_hbm, out_hbm, idx_v, hist_v):
        cid = lax.axis_index("c")
        sid = lax.axis_index("s")
        wid = cid * NUM_SUBCORES + sid

        zeros16 = jnp.zeros((16,), jnp.float32)

        @pl.loop(0, HIST, step=16)
        def _(i):
            hist_v[pl.ds(i, 16)] = zeros16

        pltpu.sync_copy(dst_hbm.at[wid], idx_v)

        ones16 = jnp.ones((16,), jnp.float32)

        @pl.loop(0, NCHUNK)
        def _(j):
            @pl.loop(0, CH, step=16)
            def _(k):
                idx = idx_v.at[j][pl.ds(k, 16)]
                plsc.addupdate_scatter(hist_v, [idx], ones16)

        pltpu.sync_copy(hist_v, out_hbm.at[wid])

    return deg_kernel(dst3d)


def _sc_scatter(g2, src3d, dst3d):
    @functools.partial(
        pl.kernel,
        out_type=jax.ShapeDtypeStruct((NUM_CORES, ACC_ROWS, CCOL), jnp.float32),
        mesh=_vector_mesh(),
        compiler_params=_sc_compiler_params(),
        scratch_types=[
            pltpu.VMEM((NCHUNK_MP, CH), jnp.int32),
            pltpu.VMEM((NCHUNK_MP, CH), jnp.int32),
            pltpu.VMEM((NBUF, CH, CCOL), jnp.float32),
            pltpu.VMEM_SHARED((ACC_ROWS, CCOL), jnp.float32),
            pltpu.VMEM_SHARED((ACC_ROWS, CCOL), jnp.float32),
            pltpu.SemaphoreType.DMA((NBUF,)),
            pltpu.SemaphoreType.DMA((NBUF,)),
        ],
    )
    def mp_kernel(g_hbm, src_hbm, dst_hbm, out_hbm,
                  src_v, dst_v, rows_v, acc_sh, g_sh, semg, sems):
        cid = lax.axis_index("c")
        sid = lax.axis_index("s")

        zeros16 = jnp.zeros((16,), jnp.float32)

        @pl.loop(0, CH)
        def _(r):
            @pl.loop(0, CCOL, step=16)
            def _(k):
                rows_v.at[0][r, pl.ds(k, 16)] = zeros16

        @pl.loop(0, NPIECE)
        def _(q):
            pltpu.sync_copy(rows_v.at[0],
                            acc_sh.at[pl.ds(sid * SLAB + q * CH, CH)])

        @pl.loop(0, NPIECE)
        def _(q):
            base = sid * SLAB + q * CH
            pltpu.sync_copy(g_hbm.at[cid].at[pl.ds(base, CH)],
                            g_sh.at[pl.ds(base, CH)])

        plsc.subcore_barrier()

        pltpu.sync_copy(src_hbm.at[sid], src_v)
        pltpu.sync_copy(dst_hbm.at[sid], dst_v)

        for b in range(AHEAD):
            pltpu.async_copy(g_sh.at[src_v.at[b]], rows_v.at[b], semg.at[b])

        for c in range(NBUF):
            pltpu.make_async_copy(
                g_sh.at[src_v.at[c]], rows_v.at[c], semg.at[c]).wait()
            pltpu.async_copy(
                rows_v.at[c], acc_sh.at[dst_v.at[c]], sems.at[c], add=True)
            n = c + AHEAD
            nb = n % NBUF
            if n >= NBUF:
                pltpu.make_async_copy(
                    rows_v.at[nb], acc_sh.at[dst_v.at[0]], sems.at[nb]).wait()
            pltpu.async_copy(g_sh.at[src_v.at[n]], rows_v.at[nb], semg.at[nb])

        @pl.loop(NBUF, NCHUNK_MP, step=NBUF)
        def _(j):
            for b in range(NBUF):
                nb = (b + AHEAD) % NBUF
                pltpu.make_async_copy(
                    g_sh.at[src_v.at[j + b]], rows_v.at[b], semg.at[b]).wait()
                pltpu.async_copy(
                    rows_v.at[b], acc_sh.at[dst_v.at[j + b]], sems.at[b],
                    add=True)
                pltpu.make_async_copy(
                    rows_v.at[nb], acc_sh.at[dst_v.at[0]], sems.at[nb]).wait()

                @pl.when(j + b + AHEAD < NCHUNK_MP)
                def _(j=j, b=b, nb=nb):
                    pltpu.async_copy(
                        g_sh.at[src_v.at[j + b + AHEAD]], rows_v.at[nb],
                        semg.at[nb])

        for b in range(NBUF - AHEAD, NBUF):
            pltpu.make_async_copy(
                rows_v.at[b], acc_sh.at[dst_v.at[0]], sems.at[b]).wait()

        plsc.subcore_barrier()

        @pl.loop(0, NPIECE)
        def _(q):
            base = sid * SLAB + q * CH
            pltpu.sync_copy(acc_sh.at[pl.ds(base, CH)],
                            out_hbm.at[cid].at[pl.ds(base, CH)])

    return mp_kernel(g2, src3d, dst3d)


def _tc_matmul(x, W):
    def body(x_ref, w_ref, o_ref):
        o_ref[...] = jnp.dot(x_ref[...], w_ref[...],
                             preferred_element_type=jnp.float32)

    nblk = N_NODES // ROWS_PER_MM_BLOCK
    return pl.pallas_call(
        body,
        out_shape=jax.ShapeDtypeStruct((N_NODES, DIM_OUT), jnp.float32),
        grid=(nblk,),
        in_specs=[
            pl.BlockSpec((ROWS_PER_MM_BLOCK, DIM_IN), lambda i: (i, 0)),
            pl.BlockSpec((DIM_IN, DIM_OUT), lambda i: (0, 0)),
        ],
        out_specs=pl.BlockSpec((ROWS_PER_MM_BLOCK, DIM_OUT), lambda i: (i, 0)),
    )(x, W)


def _tc_degsum(partials):
    def body(p_ref, o_ref):
        o_ref[...] = jnp.sum(p_ref[...], axis=0, keepdims=True) + 2.0

    return pl.pallas_call(
        body,
        out_shape=jax.ShapeDtypeStruct((1, HIST), jnp.float32),
    )(partials)


def _tc_scale(h, deg_col):
    def body(h_ref, d_ref, o_ref):
        o_ref[...] = h_ref[...] * lax.rsqrt(d_ref[...])

    nblk = N_NODES // ROWS_PER_MM_BLOCK
    return pl.pallas_call(
        body,
        out_shape=jax.ShapeDtypeStruct((N_NODES, DIM_OUT), jnp.float32),
        grid=(nblk,),
        in_specs=[
            pl.BlockSpec((ROWS_PER_MM_BLOCK, DIM_OUT), lambda i: (i, 0)),
            pl.BlockSpec((ROWS_PER_MM_BLOCK, 1), lambda i: (i, 0)),
        ],
        out_specs=pl.BlockSpec((ROWS_PER_MM_BLOCK, DIM_OUT), lambda i: (i, 0)),
    )(h, deg_col)


def _tc_final(a0, a1, g, deg_col, b2, lw2, lb2):
    def body(a0_ref, a1_ref, g_ref, d_ref, b_ref, lw_ref, lb_ref, o_ref):
        dis = lax.rsqrt(d_ref[...])
        acc = jnp.concatenate([a0_ref[...], a1_ref[...]], axis=1)
        out = dis * (acc + 2.0 * g_ref[...]) + b_ref[...]
        out = jnp.where(out >= 0, out, 0.01 * out)
        mu = jnp.mean(out, axis=1, keepdims=True)
        cen = out - mu
        var = jnp.mean(cen * cen, axis=1, keepdims=True)
        o_ref[...] = cen * lax.rsqrt(var + 1e-5) * lw_ref[...] + lb_ref[...]

    nblk = N_NODES // ROWS_PER_MM_BLOCK
    row_spec = pl.BlockSpec((ROWS_PER_MM_BLOCK, DIM_OUT), lambda i: (i, 0))
    half_spec = pl.BlockSpec((ROWS_PER_MM_BLOCK, CCOL), lambda i: (i, 0))
    vec_spec = pl.BlockSpec((1, DIM_OUT), lambda i: (0, 0))
    return pl.pallas_call(
        body,
        out_shape=jax.ShapeDtypeStruct((N_NODES, DIM_OUT), jnp.float32),
        grid=(nblk,),
        in_specs=[half_spec, half_spec, row_spec,
                  pl.BlockSpec((ROWS_PER_MM_BLOCK, 1), lambda i: (i, 0)),
                  vec_spec, vec_spec, vec_spec],
        out_specs=row_spec,
    )(a0, a1, g, deg_col, b2, lw2, lb2)


@jax.jit
def _run(x, edge_index, W, b, ln_w, ln_b):
    src = edge_index[0]
    dst = edge_index[1]
    pad = jnp.full((PAD_EDGES,), DUMMY, jnp.int32)
    src_flat = jnp.concatenate([src, pad])
    dst_flat = jnp.concatenate([dst, pad])
    dst3d = dst_flat.reshape(NW, NCHUNK, CH)
    src3d_mp = src_flat.reshape(NUM_SUBCORES, NCHUNK_MP, CH)
    dst3d_mp = dst_flat.reshape(NUM_SUBCORES, NCHUNK_MP, CH)

    partials = _sc_degree(dst3d)
    h = _tc_matmul(x, W)
    deg_col = _tc_degsum(partials).reshape(HIST, 1)[:N_NODES]
    g = _tc_scale(h, deg_col)
    g_pad = jnp.pad(g, ((0, ACC_ROWS - N_NODES), (0, 0)))
    g2 = jnp.stack([g_pad[:, :CCOL], g_pad[:, CCOL:]], axis=0)
    acc = _sc_scatter(g2, src3d_mp, dst3d_mp)

    b2 = b.reshape(1, DIM_OUT)
    lw2 = ln_w.reshape(1, DIM_OUT)
    lb2 = ln_b.reshape(1, DIM_OUT)
    return _tc_final(acc[0], acc[1], g, deg_col, b2, lw2, lb2)


def kernel(x, edge_index, W, b, ln_w, ln_b):
    return _run(x, edge_index, W, b, ln_w, ln_b)
